# Initial kernel scaffold; baseline (speedup 1.0000x reference)
import jax
import jax.numpy as jnp
from jax.experimental import pallas as pl


def kernel(x, edge_index, W1, b1, W2, b2, W3, b3, W4, b4):
    raise NotImplementedError("write your pallas kernel here")


# speedup vs baseline: 4.9350x; 4.9350x over previous
import functools

import jax
import jax.numpy as jnp
from jax import lax
from jax.experimental import pallas as pl
from jax.experimental.pallas import tpu as pltpu
from jax.experimental.pallas import tpu_sc as plsc

N = 10000
E = 160000
LANE = 128
NCORE = 2
NSUB = 16
EB = 128
EPAD = 163840
EPT = EPAD // NSUB
NBT = EPT // EB
EPW = EPAD // (NCORE * NSUB)
NBW = EPW // EB
NDRAIN = 10
RPT = N // NDRAIN
RZ = 200
ACC_ROWS = N + 8
RB = 2000
GRID = N // RB

_MESH = dict(core_axis_name="c", subcore_axis_name="s",
             num_cores=NCORE, num_subcores=NSUB)


def _make_deg():
    @functools.partial(
        pl.kernel,
        out_type=jax.ShapeDtypeStruct((NCORE, N, LANE), jnp.float32),
        mesh=plsc.VectorSubcoreMesh(**_MESH),
        scratch_types=[
            pltpu.VMEM((EB,), jnp.int32),
            pltpu.VMEM((EB, LANE), jnp.float32),
            pltpu.VMEM((RZ, LANE), jnp.float32),
            pltpu.VMEM_SHARED((ACC_ROWS, LANE), jnp.float32),
        ],
    )
    def deg(dst_hbm, ones_hbm, zeros_hbm, out_hbm, dst_v, ones_v, zero_v, acc_sh):
        core = lax.axis_index("c")
        sid = lax.axis_index("s")
        pltpu.sync_copy(ones_hbm, ones_v)
        pltpu.sync_copy(zeros_hbm, zero_v)

        @pl.when(sid < NDRAIN)
        def _():
            for z in range(RPT // RZ):
                pltpu.sync_copy(zero_v, acc_sh.at[pl.ds(sid * RPT + z * RZ, RZ), :])

        plsc.subcore_barrier()
        ebase = (core * NSUB + sid) * EPW

        @pl.loop(0, NBW)
        def _(k):
            pltpu.sync_copy(dst_hbm.at[pl.ds(ebase + k * EB, EB)], dst_v)
            pltpu.sync_copy(ones_v, acc_sh.at[dst_v], add=True)

        plsc.subcore_barrier()

        @pl.when(sid < NDRAIN)
        def _():
            pltpu.sync_copy(acc_sh.at[pl.ds(sid * RPT, RPT), :],
                            out_hbm.at[core, pl.ds(sid * RPT, RPT), :])

    return deg


def _make_agg(nc):
    ncs = nc // NCORE

    @functools.partial(
        pl.kernel,
        out_type=jax.ShapeDtypeStruct((nc, N, LANE), jnp.float32),
        mesh=plsc.VectorSubcoreMesh(**_MESH),
        scratch_types=[
            pltpu.VMEM((EB,), jnp.int32),
            pltpu.VMEM((EB,), jnp.int32),
            pltpu.VMEM((EB,), jnp.int32),
            pltpu.VMEM((EB, LANE), jnp.float32),
            pltpu.VMEM((RZ, LANE), jnp.float32),
            pltpu.VMEM_SHARED((ACC_ROWS, LANE), jnp.float32),
            pltpu.SemaphoreType.DMA,
        ],
    )
    def agg(zs_hbm, src_hbm, dst_hbm, zeros_hbm, out_hbm,
            src_v, gidx_v, dst_v, rows_v, zero_v, acc_sh, sem):
        core = lax.axis_index("c")
        sid = lax.axis_index("s")
        ebase = sid * EPT
        pltpu.sync_copy(zeros_hbm, zero_v)
        for j in range(ncs):
            cg = core * ncs + j
            off = cg * N
            @pl.when(sid < NDRAIN)
            def _():
                for z in range(RPT // RZ):
                    pltpu.sync_copy(zero_v, acc_sh.at[pl.ds(sid * RPT + z * RZ, RZ), :])

            plsc.subcore_barrier()

            @pl.loop(0, NBT)
            def _(k):
                b0 = ebase + k * EB
                pltpu.sync_copy(src_hbm.at[pl.ds(b0, EB)], src_v)
                pltpu.sync_copy(dst_hbm.at[pl.ds(b0, EB)], dst_v)
                for i in range(EB // 16):
                    gidx_v[pl.ds(i * 16, 16)] = src_v[pl.ds(i * 16, 16)] + off
                pltpu.async_copy(zs_hbm.at[gidx_v], rows_v, sem).wait()
                pltpu.sync_copy(rows_v, acc_sh.at[dst_v], add=True)

            plsc.subcore_barrier()

            @pl.when(sid < NDRAIN)
            def _():
                pltpu.sync_copy(acc_sh.at[pl.ds(sid * RPT, RPT), :],
                                out_hbm.at[cg, pl.ds(sid * RPT, RPT), :])

    return agg


def _full(shape):
    return pl.BlockSpec(shape, lambda i: tuple(0 for _ in shape))


def _prep(degpart, x):
    def body(deg_ref, x_ref, dis_ref, xs_ref):
        d = deg_ref[0] + deg_ref[1] + 1.0
        dis = lax.rsqrt(d)
        dis_ref[...] = dis
        for c in range(2):
            xs_ref[c] = x_ref[:, c * LANE:(c + 1) * LANE] * dis

    return pl.pallas_call(
        body,
        grid=(GRID,),
        in_specs=[pl.BlockSpec((2, RB, LANE), lambda i: (0, i, 0)),
                  pl.BlockSpec((RB, 2 * LANE), lambda i: (i, 0))],
        out_specs=[pl.BlockSpec((RB, LANE), lambda i: (i, 0)),
                   pl.BlockSpec((2, RB, LANE), lambda i: (0, i, 0))],
        out_shape=[jax.ShapeDtypeStruct((N, LANE), jnp.float32),
                   jax.ShapeDtypeStruct((2, N, LANE), jnp.float32)],
    )(degpart, x)


def _layer12(acc, xs, dis, W1, b1, W2):
    def body(acc_ref, xs_ref, dis_ref, w1_ref, b1_ref, w2_ref, out_ref):
        dis = dis_ref[...]
        p = jnp.concatenate(
            [dis * (acc_ref[c] + xs_ref[c]) for c in range(2)], axis=1)
        h = jnp.dot(p, w1_ref[...], preferred_element_type=jnp.float32)
        h = jnp.maximum(h + b1_ref[...], 0.0)
        z = jnp.dot(h, w2_ref[...], preferred_element_type=jnp.float32)
        for c in range(4):
            out_ref[c] = z[:, c * LANE:(c + 1) * LANE] * dis

    return pl.pallas_call(
        body,
        grid=(GRID,),
        in_specs=[pl.BlockSpec((2, RB, LANE), lambda i: (0, i, 0)),
                  pl.BlockSpec((2, RB, LANE), lambda i: (0, i, 0)),
                  pl.BlockSpec((RB, LANE), lambda i: (i, 0)),
                  _full(W1.shape), _full(b1.shape), _full(W2.shape)],
        out_specs=pl.BlockSpec((4, RB, LANE), lambda i: (0, i, 0)),
        out_shape=jax.ShapeDtypeStruct((4, N, LANE), jnp.float32),
    )(acc, xs, dis, W1, b1, W2)


def _layer_mid(acc, zs, dis, b, W, nc_out):
    def body(acc_ref, zs_ref, dis_ref, b_ref, w_ref, out_ref):
        dis = dis_ref[...]
        h = jnp.concatenate(
            [jnp.maximum(dis * (acc_ref[c] + zs_ref[c])
                         + b_ref[:, c * LANE:(c + 1) * LANE], 0.0)
             for c in range(4)], axis=1)
        z = jnp.dot(h, w_ref[...], preferred_element_type=jnp.float32)
        for c in range(nc_out):
            out_ref[c] = z[:, c * LANE:(c + 1) * LANE] * dis

    return pl.pallas_call(
        body,
        grid=(GRID,),
        in_specs=[pl.BlockSpec((4, RB, LANE), lambda i: (0, i, 0)),
                  pl.BlockSpec((4, RB, LANE), lambda i: (0, i, 0)),
                  pl.BlockSpec((RB, LANE), lambda i: (i, 0)),
                  _full(b.shape), _full(W.shape)],
        out_specs=pl.BlockSpec((nc_out, RB, LANE), lambda i: (0, i, 0)),
        out_shape=jax.ShapeDtypeStruct((nc_out, N, LANE), jnp.float32),
    )(acc, zs, dis, b, W)


def _final(acc, zs, dis, b4):
    def body(acc_ref, zs_ref, dis_ref, b_ref, out_ref):
        dis = dis_ref[...]
        for c in range(2):
            out_ref[:, c * LANE:(c + 1) * LANE] = (
                dis * (acc_ref[c] + zs_ref[c]) + b_ref[:, c * LANE:(c + 1) * LANE])

    return pl.pallas_call(
        body,
        grid=(GRID,),
        in_specs=[pl.BlockSpec((2, RB, LANE), lambda i: (0, i, 0)),
                  pl.BlockSpec((2, RB, LANE), lambda i: (0, i, 0)),
                  pl.BlockSpec((RB, LANE), lambda i: (i, 0)),
                  _full(b4.shape)],
        out_specs=pl.BlockSpec((RB, 2 * LANE), lambda i: (i, 0)),
        out_shape=jax.ShapeDtypeStruct((N, 2 * LANE), jnp.float32),
    )(acc, zs, dis, b4)


@functools.lru_cache(maxsize=None)
def _deg_fn():
    return _make_deg()


@functools.lru_cache(maxsize=None)
def _agg_fn(nc):
    return _make_agg(nc)


def kernel(x, edge_index, W1, b1, W2, b2, W3, b3, W4, b4):
    src = edge_index[0].astype(jnp.int32)
    dst = edge_index[1].astype(jnp.int32)
    pad = EPAD - E
    src_p = jnp.concatenate([src, jnp.zeros((pad,), jnp.int32)])
    dst_p = jnp.concatenate([dst, jnp.full((pad,), N, jnp.int32)])
    ones_in = jnp.ones((EB, LANE), jnp.float32)
    zeros_in = jnp.zeros((RZ, LANE), jnp.float32)

    deg, agg2, agg4 = _deg_fn(), _agg_fn(2), _agg_fn(4)
    degpart = deg(dst_p, ones_in, zeros_in)
    dis, xs = _prep(degpart, x)
    acc = agg2(xs.reshape(2 * N, LANE), src_p, dst_p, zeros_in)
    z2s = _layer12(acc, xs, dis, W1, b1.reshape(1, -1), W2)
    acc = agg4(z2s.reshape(4 * N, LANE), src_p, dst_p, zeros_in)
    z3s = _layer_mid(acc, z2s, dis, b2.reshape(1, -1), W3, 4)
    acc = agg4(z3s.reshape(4 * N, LANE), src_p, dst_p, zeros_in)
    z4s = _layer_mid(acc, z3s, dis, b3.reshape(1, -1), W4, 2)
    acc = agg2(z4s.reshape(2 * N, LANE), src_p, dst_p, zeros_in)
    return _final(acc, z4s, dis, b4.reshape(1, -1))


# speedup vs baseline: 6.4332x; 1.3036x over previous
import functools

import jax
import jax.numpy as jnp
from jax import lax
from jax.experimental import pallas as pl
from jax.experimental.pallas import tpu as pltpu
from jax.experimental.pallas import tpu_sc as plsc

N = 10000
E = 160000
LANE = 128
NCORE = 2
NSUB = 16
EB = 128
EPAD = 163840
EPT = EPAD // NSUB
NBT = EPT // EB
EPW = EPAD // (NCORE * NSUB)
NBW = EPW // EB
NDRAIN = 10
RPT = N // NDRAIN
RZ = 200
NBUF = 2
NB_H = NBT // 2
ACC_ROWS = N + 8
RB = 2000
GRID = N // RB

_MESH = dict(core_axis_name="c", subcore_axis_name="s",
             num_cores=NCORE, num_subcores=NSUB)


def _make_deg():
    @functools.partial(
        pl.kernel,
        out_type=jax.ShapeDtypeStruct((NCORE, N, LANE), jnp.float32),
        mesh=plsc.VectorSubcoreMesh(**_MESH),
        scratch_types=[
            pltpu.VMEM((NBW, EB), jnp.int32),
            pltpu.VMEM((EB, LANE), jnp.float32),
            pltpu.VMEM_SHARED((ACC_ROWS, LANE), jnp.float32),
            pltpu.SemaphoreType.DMA,
        ],
    )
    def deg(dst2_hbm, ones_hbm, zeros_hbm, out_hbm, dst_s, ones_v, acc_sh, sem):
        core = lax.axis_index("c")
        sid = lax.axis_index("s")
        w = core * NSUB + sid
        pltpu.sync_copy(dst2_hbm.at[pl.ds(w * NBW, NBW), :], dst_s)
        pltpu.sync_copy(ones_hbm, ones_v)

        @pl.when(sid < NDRAIN)
        def _():
            for z in range(RPT // RZ):
                pltpu.sync_copy(zeros_hbm, acc_sh.at[pl.ds(sid * RPT + z * RZ, RZ), :])

        plsc.subcore_barrier()

        @pl.loop(0, NBW)
        def _(k):
            pltpu.async_copy(ones_v, acc_sh.at[dst_s.at[k]], sem, add=True)

        @pl.loop(0, NBW)
        def _(k):
            pltpu.make_async_copy(ones_v, acc_sh.at[dst_s.at[k]], sem).wait()

        plsc.subcore_barrier()

        @pl.when(sid < NDRAIN)
        def _():
            pltpu.sync_copy(acc_sh.at[pl.ds(sid * RPT, RPT), :],
                            out_hbm.at[core, pl.ds(sid * RPT, RPT), :])

    return deg


def _make_agg(nc):
    ncs = nc // NCORE

    @functools.partial(
        pl.kernel,
        out_type=jax.ShapeDtypeStruct((nc, N, LANE), jnp.float32),
        mesh=plsc.VectorSubcoreMesh(**_MESH),
        scratch_types=[
            pltpu.VMEM((NB_H, EB), jnp.int32),
            pltpu.VMEM((NB_H, EB), jnp.int32),
            pltpu.VMEM((NB_H, EB), jnp.int32),
        ] + [pltpu.VMEM((EB, LANE), jnp.float32) for _ in range(NBUF)]
          + [pltpu.SemaphoreType.DMA for _ in range(2 * NBUF)]
          + [pltpu.VMEM_SHARED((ACC_ROWS, LANE), jnp.float32)],
    )
    def agg(zs_hbm, src2_hbm, dst2_hbm, zeros_hbm, out_hbm,
            src_s, gidx_s, dst_s, r0, r1, g0, g1, s0, s1, acc_sh):
        rows = (r0, r1)
        gsem = (g0, g1)
        ssem = (s0, s1)
        core = lax.axis_index("c")
        sid = lax.axis_index("s")
        tb = sid * NBT

        def start_g(k, b):
            pltpu.async_copy(zs_hbm.at[gidx_s.at[k]], rows[b], gsem[b])

        def wait_g(k, b):
            pltpu.make_async_copy(zs_hbm.at[gidx_s.at[k]], rows[b], gsem[b]).wait()

        def start_s(k, b):
            pltpu.async_copy(rows[b], acc_sh.at[dst_s.at[k]], ssem[b], add=True)

        def wait_s(k, b):
            pltpu.make_async_copy(rows[b], acc_sh.at[dst_s.at[k]], ssem[b]).wait()

        for j in range(ncs):
            cg = core * ncs + j
            off = cg * N

            @pl.when(sid < NDRAIN)
            def _():
                for z in range(RPT // RZ):
                    pltpu.sync_copy(zeros_hbm, acc_sh.at[pl.ds(sid * RPT + z * RZ, RZ), :])

            plsc.subcore_barrier()

            for h in range(2):
                hb = tb + h * NB_H
                pltpu.sync_copy(src2_hbm.at[pl.ds(hb, NB_H), :], src_s)
                pltpu.sync_copy(dst2_hbm.at[pl.ds(hb, NB_H), :], dst_s)

                @pl.loop(0, NB_H)
                def _(r):
                    for i in range(EB // 16):
                        gidx_s[r, pl.ds(i * 16, 16)] = src_s[r, pl.ds(i * 16, 16)] + off

                start_g(0, 0)

                @pl.loop(0, NB_H // NBUF)
                def _(grp):
                    base = grp * NBUF
                    for b in range(NBUF):
                        k = base + b
                        bn = (b + 1) % NBUF
                        wait_g(k, b)
                        start_s(k, b)

                        @pl.when(k + 1 < NB_H)
                        def _():
                            @pl.when(k >= 1)
                            def _():
                                wait_s(k - 1, bn)

                            start_g(k + 1, bn)

                wait_s(NB_H - 2, 0)
                wait_s(NB_H - 1, 1)

            plsc.subcore_barrier()

            @pl.when(sid < NDRAIN)
            def _():
                pltpu.sync_copy(acc_sh.at[pl.ds(sid * RPT, RPT), :],
                                out_hbm.at[cg, pl.ds(sid * RPT, RPT), :])

    return agg


def _full(shape):
    return pl.BlockSpec(shape, lambda i: tuple(0 for _ in shape))


def _prep(degpart, x):
    def body(deg_ref, x_ref, dis_ref, xs_ref):
        d = deg_ref[0] + deg_ref[1] + 1.0
        dis = lax.rsqrt(d)
        dis_ref[...] = dis
        for c in range(2):
            xs_ref[c] = x_ref[:, c * LANE:(c + 1) * LANE] * dis

    return pl.pallas_call(
        body,
        grid=(GRID,),
        in_specs=[pl.BlockSpec((2, RB, LANE), lambda i: (0, i, 0)),
                  pl.BlockSpec((RB, 2 * LANE), lambda i: (i, 0))],
        out_specs=[pl.BlockSpec((RB, LANE), lambda i: (i, 0)),
                   pl.BlockSpec((2, RB, LANE), lambda i: (0, i, 0))],
        out_shape=[jax.ShapeDtypeStruct((N, LANE), jnp.float32),
                   jax.ShapeDtypeStruct((2, N, LANE), jnp.float32)],
    )(degpart, x)


def _layer12(acc, xs, dis, W1, b1, W2):
    def body(acc_ref, xs_ref, dis_ref, w1_ref, b1_ref, w2_ref, out_ref):
        dis = dis_ref[...]
        p = jnp.concatenate(
            [dis * (acc_ref[c] + xs_ref[c]) for c in range(2)], axis=1)
        h = jnp.dot(p, w1_ref[...], preferred_element_type=jnp.float32)
        h = jnp.maximum(h + b1_ref[...], 0.0)
        z = jnp.dot(h, w2_ref[...], preferred_element_type=jnp.float32)
        for c in range(4):
            out_ref[c] = z[:, c * LANE:(c + 1) * LANE] * dis

    return pl.pallas_call(
        body,
        grid=(GRID,),
        in_specs=[pl.BlockSpec((2, RB, LANE), lambda i: (0, i, 0)),
                  pl.BlockSpec((2, RB, LANE), lambda i: (0, i, 0)),
                  pl.BlockSpec((RB, LANE), lambda i: (i, 0)),
                  _full(W1.shape), _full(b1.shape), _full(W2.shape)],
        out_specs=pl.BlockSpec((4, RB, LANE), lambda i: (0, i, 0)),
        out_shape=jax.ShapeDtypeStruct((4, N, LANE), jnp.float32),
    )(acc, xs, dis, W1, b1, W2)


def _layer_mid(acc, zs, dis, b, W, nc_out):
    def body(acc_ref, zs_ref, dis_ref, b_ref, w_ref, out_ref):
        dis = dis_ref[...]
        h = jnp.concatenate(
            [jnp.maximum(dis * (acc_ref[c] + zs_ref[c])
                         + b_ref[:, c * LANE:(c + 1) * LANE], 0.0)
             for c in range(4)], axis=1)
        z = jnp.dot(h, w_ref[...], preferred_element_type=jnp.float32)
        for c in range(nc_out):
            out_ref[c] = z[:, c * LANE:(c + 1) * LANE] * dis

    return pl.pallas_call(
        body,
        grid=(GRID,),
        in_specs=[pl.BlockSpec((4, RB, LANE), lambda i: (0, i, 0)),
                  pl.BlockSpec((4, RB, LANE), lambda i: (0, i, 0)),
                  pl.BlockSpec((RB, LANE), lambda i: (i, 0)),
                  _full(b.shape), _full(W.shape)],
        out_specs=pl.BlockSpec((nc_out, RB, LANE), lambda i: (0, i, 0)),
        out_shape=jax.ShapeDtypeStruct((nc_out, N, LANE), jnp.float32),
    )(acc, zs, dis, b, W)


def _final(acc, zs, dis, b4):
    def body(acc_ref, zs_ref, dis_ref, b_ref, out_ref):
        dis = dis_ref[...]
        for c in range(2):
            out_ref[:, c * LANE:(c + 1) * LANE] = (
                dis * (acc_ref[c] + zs_ref[c]) + b_ref[:, c * LANE:(c + 1) * LANE])

    return pl.pallas_call(
        body,
        grid=(GRID,),
        in_specs=[pl.BlockSpec((2, RB, LANE), lambda i: (0, i, 0)),
                  pl.BlockSpec((2, RB, LANE), lambda i: (0, i, 0)),
                  pl.BlockSpec((RB, LANE), lambda i: (i, 0)),
                  _full(b4.shape)],
        out_specs=pl.BlockSpec((RB, 2 * LANE), lambda i: (i, 0)),
        out_shape=jax.ShapeDtypeStruct((N, 2 * LANE), jnp.float32),
    )(acc, zs, dis, b4)


@functools.lru_cache(maxsize=None)
def _deg_fn():
    return _make_deg()


@functools.lru_cache(maxsize=None)
def _agg_fn(nc):
    return _make_agg(nc)


def kernel(x, edge_index, W1, b1, W2, b2, W3, b3, W4, b4):
    src = edge_index[0].astype(jnp.int32)
    dst = edge_index[1].astype(jnp.int32)
    pad = EPAD - E
    src_p = jnp.concatenate([src, jnp.zeros((pad,), jnp.int32)])
    dst_p = jnp.concatenate([dst, jnp.full((pad,), N, jnp.int32)])
    ones_in = jnp.ones((EB, LANE), jnp.float32)
    zeros_in = jnp.zeros((RZ, LANE), jnp.float32)

    deg, agg2, agg4 = _deg_fn(), _agg_fn(2), _agg_fn(4)
    src2 = src_p.reshape(EPAD // EB, EB)
    dst2 = dst_p.reshape(EPAD // EB, EB)
    degpart = deg(dst2, ones_in, zeros_in)
    dis, xs = _prep(degpart, x)
    acc = agg2(xs.reshape(2 * N, LANE), src2, dst2, zeros_in)
    z2s = _layer12(acc, xs, dis, W1, b1.reshape(1, -1), W2)
    acc = agg4(z2s.reshape(4 * N, LANE), src2, dst2, zeros_in)
    z3s = _layer_mid(acc, z2s, dis, b2.reshape(1, -1), W3, 4)
    acc = agg4(z3s.reshape(4 * N, LANE), src2, dst2, zeros_in)
    z4s = _layer_mid(acc, z3s, dis, b3.reshape(1, -1), W4, 2)
    acc = agg2(z4s.reshape(2 * N, LANE), src2, dst2, zeros_in)
    return _final(acc, z4s, dis, b4.reshape(1, -1))


# speedup vs baseline: 6.6810x; 1.0385x over previous
import functools

import jax
import jax.numpy as jnp
from jax import lax
from jax.experimental import pallas as pl
from jax.experimental.pallas import tpu as pltpu
from jax.experimental.pallas import tpu_sc as plsc

N = 10000
E = 160000
LANE = 128
NCORE = 2
NSUB = 16
EB = 64
EPAD = 163840
EPT = EPAD // NSUB
NBT = EPT // EB
EPW = EPAD // (NCORE * NSUB)
NBW = EPW // EB
NDRAIN = 10
RPT = N // NDRAIN
RZ = 200
NBUF = 4
NSEG = 4
NB_H = NBT // NSEG
ACC_ROWS = N + 8
RB = 2000
GRID = N // RB

_MESH = dict(core_axis_name="c", subcore_axis_name="s",
             num_cores=NCORE, num_subcores=NSUB)


def _make_deg():
    @functools.partial(
        pl.kernel,
        out_type=jax.ShapeDtypeStruct((NCORE, N, LANE), jnp.float32),
        mesh=plsc.VectorSubcoreMesh(**_MESH),
        scratch_types=[
            pltpu.VMEM((NBW, EB), jnp.int32),
            pltpu.VMEM((EB, LANE), jnp.float32),
            pltpu.VMEM_SHARED((ACC_ROWS, LANE), jnp.float32),
            pltpu.SemaphoreType.DMA,
        ],
    )
    def deg(dst2_hbm, ones_hbm, zeros_hbm, out_hbm, dst_s, ones_v, acc_sh, sem):
        core = lax.axis_index("c")
        sid = lax.axis_index("s")
        w = core * NSUB + sid
        pltpu.sync_copy(dst2_hbm.at[pl.ds(w * NBW, NBW), :], dst_s)
        pltpu.sync_copy(ones_hbm, ones_v)

        @pl.when(sid < NDRAIN)
        def _():
            for z in range(RPT // RZ):
                pltpu.sync_copy(zeros_hbm, acc_sh.at[pl.ds(sid * RPT + z * RZ, RZ), :])

        plsc.subcore_barrier()

        @pl.loop(0, NBW)
        def _(k):
            pltpu.async_copy(ones_v, acc_sh.at[dst_s.at[k]], sem, add=True)

        @pl.loop(0, NBW)
        def _(k):
            pltpu.make_async_copy(ones_v, acc_sh.at[dst_s.at[k]], sem).wait()

        plsc.subcore_barrier()

        @pl.when(sid < NDRAIN)
        def _():
            pltpu.sync_copy(acc_sh.at[pl.ds(sid * RPT, RPT), :],
                            out_hbm.at[core, pl.ds(sid * RPT, RPT), :])

    return deg


def _make_agg(nc):
    ncs = nc // NCORE

    @functools.partial(
        pl.kernel,
        out_type=jax.ShapeDtypeStruct((nc, N, LANE), jnp.float32),
        mesh=plsc.VectorSubcoreMesh(**_MESH),
        scratch_types=[
            pltpu.VMEM((NB_H, EB), jnp.int32),
            pltpu.VMEM((NB_H, EB), jnp.int32),
            pltpu.VMEM((NB_H, EB), jnp.int32),
        ] + [pltpu.VMEM((EB, LANE), jnp.float32) for _ in range(NBUF)]
          + [pltpu.SemaphoreType.DMA for _ in range(2 * NBUF)]
          + [pltpu.VMEM_SHARED((ACC_ROWS, LANE), jnp.float32)],
    )
    def agg(zs_hbm, src2_hbm, dst2_hbm, zeros_hbm, out_hbm,
            src_s, gidx_s, dst_s, r0, r1, r2, r3,
            g0, g1, g2, g3, s0, s1, s2, s3, acc_sh):
        rows = (r0, r1, r2, r3)
        gsem = (g0, g1, g2, g3)
        ssem = (s0, s1, s2, s3)
        core = lax.axis_index("c")
        sid = lax.axis_index("s")
        tb = sid * NBT

        def start_g(k, b):
            pltpu.async_copy(zs_hbm.at[gidx_s.at[k]], rows[b], gsem[b])

        def wait_g(k, b):
            pltpu.make_async_copy(zs_hbm.at[gidx_s.at[k]], rows[b], gsem[b]).wait()

        def start_s(k, b):
            pltpu.async_copy(rows[b], acc_sh.at[dst_s.at[k]], ssem[b], add=True)

        def wait_s(k, b):
            pltpu.make_async_copy(rows[b], acc_sh.at[dst_s.at[k]], ssem[b]).wait()

        for j in range(ncs):
            cg = core * ncs + j
            off = cg * N

            @pl.when(sid < NDRAIN)
            def _():
                for z in range(RPT // RZ):
                    pltpu.sync_copy(zeros_hbm, acc_sh.at[pl.ds(sid * RPT + z * RZ, RZ), :])

            plsc.subcore_barrier()

            for h in range(NSEG):
                hb = tb + h * NB_H
                pltpu.sync_copy(src2_hbm.at[pl.ds(hb, NB_H), :], src_s)
                pltpu.sync_copy(dst2_hbm.at[pl.ds(hb, NB_H), :], dst_s)

                @pl.loop(0, NB_H)
                def _(r):
                    for i in range(EB // 16):
                        gidx_s[r, pl.ds(i * 16, 16)] = src_s[r, pl.ds(i * 16, 16)] + off

                start_g(0, 0)
                start_g(1, 1)

                @pl.loop(0, NB_H // NBUF)
                def _(grp):
                    base = grp * NBUF
                    for b in range(NBUF):
                        k = base + b
                        b2 = (b + 2) % NBUF
                        wait_g(k, b)
                        start_s(k, b)

                        @pl.when(k + 2 < NB_H)
                        def _():
                            @pl.when(k >= 2)
                            def _():
                                wait_s(k - 2, b2)

                            start_g(k + 2, b2)

                for b in range(NBUF):
                    wait_s(NB_H - NBUF + b, b)

            plsc.subcore_barrier()

            @pl.when(sid < NDRAIN)
            def _():
                pltpu.sync_copy(acc_sh.at[pl.ds(sid * RPT, RPT), :],
                                out_hbm.at[cg, pl.ds(sid * RPT, RPT), :])

    return agg


def _full(shape):
    return pl.BlockSpec(shape, lambda i: tuple(0 for _ in shape))


def _prep(degpart, x):
    def body(deg_ref, x_ref, dis_ref, xs_ref):
        d = deg_ref[0] + deg_ref[1] + 1.0
        dis = lax.rsqrt(d)
        dis_ref[...] = dis
        for c in range(2):
            xs_ref[c] = x_ref[:, c * LANE:(c + 1) * LANE] * dis

    return pl.pallas_call(
        body,
        grid=(GRID,),
        in_specs=[pl.BlockSpec((2, RB, LANE), lambda i: (0, i, 0)),
                  pl.BlockSpec((RB, 2 * LANE), lambda i: (i, 0))],
        out_specs=[pl.BlockSpec((RB, LANE), lambda i: (i, 0)),
                   pl.BlockSpec((2, RB, LANE), lambda i: (0, i, 0))],
        out_shape=[jax.ShapeDtypeStruct((N, LANE), jnp.float32),
                   jax.ShapeDtypeStruct((2, N, LANE), jnp.float32)],
    )(degpart, x)


def _layer12(acc, xs, dis, W1, b1, W2):
    def body(acc_ref, xs_ref, dis_ref, w1_ref, b1_ref, w2_ref, out_ref):
        dis = dis_ref[...]
        p = jnp.concatenate(
            [dis * (acc_ref[c] + xs_ref[c]) for c in range(2)], axis=1)
        h = jnp.dot(p, w1_ref[...], preferred_element_type=jnp.float32)
        h = jnp.maximum(h + b1_ref[...], 0.0)
        z = jnp.dot(h, w2_ref[...], preferred_element_type=jnp.float32)
        for c in range(4):
            out_ref[c] = z[:, c * LANE:(c + 1) * LANE] * dis

    return pl.pallas_call(
        body,
        grid=(GRID,),
        in_specs=[pl.BlockSpec((2, RB, LANE), lambda i: (0, i, 0)),
                  pl.BlockSpec((2, RB, LANE), lambda i: (0, i, 0)),
                  pl.BlockSpec((RB, LANE), lambda i: (i, 0)),
                  _full(W1.shape), _full(b1.shape), _full(W2.shape)],
        out_specs=pl.BlockSpec((4, RB, LANE), lambda i: (0, i, 0)),
        out_shape=jax.ShapeDtypeStruct((4, N, LANE), jnp.float32),
    )(acc, xs, dis, W1, b1, W2)


def _layer_mid(acc, zs, dis, b, W, nc_out):
    def body(acc_ref, zs_ref, dis_ref, b_ref, w_ref, out_ref):
        dis = dis_ref[...]
        h = jnp.concatenate(
            [jnp.maximum(dis * (acc_ref[c] + zs_ref[c])
                         + b_ref[:, c * LANE:(c + 1) * LANE], 0.0)
             for c in range(4)], axis=1)
        z = jnp.dot(h, w_ref[...], preferred_element_type=jnp.float32)
        for c in range(nc_out):
            out_ref[c] = z[:, c * LANE:(c + 1) * LANE] * dis

    return pl.pallas_call(
        body,
        grid=(GRID,),
        in_specs=[pl.BlockSpec((4, RB, LANE), lambda i: (0, i, 0)),
                  pl.BlockSpec((4, RB, LANE), lambda i: (0, i, 0)),
                  pl.BlockSpec((RB, LANE), lambda i: (i, 0)),
                  _full(b.shape), _full(W.shape)],
        out_specs=pl.BlockSpec((nc_out, RB, LANE), lambda i: (0, i, 0)),
        out_shape=jax.ShapeDtypeStruct((nc_out, N, LANE), jnp.float32),
    )(acc, zs, dis, b, W)


def _final(acc, zs, dis, b4):
    def body(acc_ref, zs_ref, dis_ref, b_ref, out_ref):
        dis = dis_ref[...]
        for c in range(2):
            out_ref[:, c * LANE:(c + 1) * LANE] = (
                dis * (acc_ref[c] + zs_ref[c]) + b_ref[:, c * LANE:(c + 1) * LANE])

    return pl.pallas_call(
        body,
        grid=(GRID,),
        in_specs=[pl.BlockSpec((2, RB, LANE), lambda i: (0, i, 0)),
                  pl.BlockSpec((2, RB, LANE), lambda i: (0, i, 0)),
                  pl.BlockSpec((RB, LANE), lambda i: (i, 0)),
                  _full(b4.shape)],
        out_specs=pl.BlockSpec((RB, 2 * LANE), lambda i: (i, 0)),
        out_shape=jax.ShapeDtypeStruct((N, 2 * LANE), jnp.float32),
    )(acc, zs, dis, b4)


@functools.lru_cache(maxsize=None)
def _deg_fn():
    return _make_deg()


@functools.lru_cache(maxsize=None)
def _agg_fn(nc):
    return _make_agg(nc)


def kernel(x, edge_index, W1, b1, W2, b2, W3, b3, W4, b4):
    src = edge_index[0].astype(jnp.int32)
    dst = edge_index[1].astype(jnp.int32)
    pad = EPAD - E
    src_p = jnp.concatenate([src, jnp.zeros((pad,), jnp.int32)])
    dst_p = jnp.concatenate([dst, jnp.full((pad,), N, jnp.int32)])
    ones_in = jnp.ones((EB, LANE), jnp.float32)
    zeros_in = jnp.zeros((RZ, LANE), jnp.float32)

    deg, agg2, agg4 = _deg_fn(), _agg_fn(2), _agg_fn(4)
    src2 = src_p.reshape(EPAD // EB, EB)
    dst2 = dst_p.reshape(EPAD // EB, EB)
    degpart = deg(dst2, ones_in, zeros_in)
    dis, xs = _prep(degpart, x)
    acc = agg2(xs.reshape(2 * N, LANE), src2, dst2, zeros_in)
    z2s = _layer12(acc, xs, dis, W1, b1.reshape(1, -1), W2)
    acc = agg4(z2s.reshape(4 * N, LANE), src2, dst2, zeros_in)
    z3s = _layer_mid(acc, z2s, dis, b2.reshape(1, -1), W3, 4)
    acc = agg4(z3s.reshape(4 * N, LANE), src2, dst2, zeros_in)
    z4s = _layer_mid(acc, z3s, dis, b3.reshape(1, -1), W4, 2)
    acc = agg2(z4s.reshape(2 * N, LANE), src2, dst2, zeros_in)
    return _final(acc, z4s, dis, b4.reshape(1, -1))
